# speedup vs baseline: 3.6212x; 3.6212x over previous
import jax
import jax.numpy as jnp
from jax.experimental import pallas as pl

B, S, E, KD, VD, H, O = 64, 2048, 1024, 512, 512, 16, 20
DH = E // H
F32 = jnp.float32


def _setup_body(node_ref, Wlin_ref, blin_ref, Wq_ref, bq_ref, Wk_ref,
                p_ref, pred0_ref):
    x = node_ref[...]
    pred0_ref[...] = (
        jnp.dot(x, Wlin_ref[...].T, preferred_element_type=F32)
        + blin_ref[...][None, :])
    q = jnp.dot(x, Wq_ref[...].T, preferred_element_type=F32) + bq_ref[...][None, :]
    q = q * jnp.float32(1.0 / 8.0)
    for h in range(H):
        qh = q[:, h * DH:(h + 1) * DH]
        wk_h = Wk_ref[h * DH:(h + 1) * DH, :]
        p_ref[:, h, :] = jnp.dot(qh, wk_h, preferred_element_type=F32)


def _attn_body(p_ref, amask_ref, kpm_ref, seq_ref, u_ref):
    seq = seq_ref[0]
    pb = p_ref[0]
    scores = jax.lax.dot_general(
        seq, pb, (((1,), (1,)), ((), ())),
        preferred_element_type=F32)
    am = amask_ref[0, 0, :]
    kpm = kpm_ref[0, 0, :]
    scores = scores + am[:, None]
    scores = jnp.where(kpm[:, None] > 0, jnp.float32(-1e9), scores)
    m = jnp.max(scores, axis=0, keepdims=True)
    e = jnp.exp(scores - m)
    l = jnp.sum(e, axis=0, keepdims=True)
    a = e / l
    u_ref[0] = jax.lax.dot_general(
        a, seq, (((0,), (0,)), ((), ())),
        preferred_element_type=F32)


def _final_body(u_ref, Wv_ref, bv_ref, Wo_ref, bo_ref, Wla_ref, bla_ref,
                pred0_ref, g_ref, mask_ref, onehot_ref, pred_ref):
    ctx_parts = []
    for h in range(H):
        uh = u_ref[:, h, :]
        wv_h = Wv_ref[h * DH:(h + 1) * DH, :]
        ctx_parts.append(jax.lax.dot_general(
            uh, wv_h, (((1,), (1,)), ((), ())),
            preferred_element_type=F32))
    ctx = jnp.concatenate(ctx_parts, axis=1) + bv_ref[...][None, :]
    hout = jnp.dot(ctx, Wo_ref[...].T, preferred_element_type=F32) + bo_ref[...][None, :]
    pred = (pred0_ref[...]
            + jnp.dot(hout, Wla_ref[...].T, preferred_element_type=F32)
            + bla_ref[...][None, :])
    y = pred + g_ref[...]
    top = jnp.argmax(y, axis=-1)
    onehot = (jax.lax.broadcasted_iota(jnp.int32, (B, O), 1)
              == top[:, None]).astype(F32)
    msk = mask_ref[...]
    pred_ref[...] = pred * msk
    onehot_ref[...] = onehot * msk


def kernel(node_emb, seq_emb, mask, key_padding_mask, attn_mask,
           W_lin, b_lin, Wq, Wk, Wv, bq, bk, bv, Wo, bo, W_la, b_la):
    del bk
    x = node_emb.reshape(B, E)
    kpm_f = key_padding_mask.astype(F32).reshape(B, 1, S)
    amask = attn_mask.reshape(B, 1, S)
    mask2d = mask.reshape(B, 1)

    p, pred0 = pl.pallas_call(
        _setup_body,
        out_shape=(
            jax.ShapeDtypeStruct((B, H, KD), F32),
            jax.ShapeDtypeStruct((B, O), F32),
        ),
    )(x, W_lin, b_lin, Wq, bq, Wk)

    u = pl.pallas_call(
        _attn_body,
        grid=(B,),
        in_specs=[
            pl.BlockSpec((1, H, KD), lambda b: (b, 0, 0)),
            pl.BlockSpec((1, 1, S), lambda b: (b, 0, 0)),
            pl.BlockSpec((1, 1, S), lambda b: (b, 0, 0)),
            pl.BlockSpec((1, S, KD), lambda b: (b, 0, 0)),
        ],
        out_specs=pl.BlockSpec((1, H, KD), lambda b: (b, 0, 0)),
        out_shape=jax.ShapeDtypeStruct((B, H, KD), F32),
    )(p, amask, kpm_f, seq_emb)

    skey = jax.random.fold_in(jax.random.key(0), 12345)
    g = jax.random.gumbel(skey, (B, O), F32)

    onehot, pred = pl.pallas_call(
        _final_body,
        out_shape=(
            jax.ShapeDtypeStruct((B, O), F32),
            jax.ShapeDtypeStruct((B, O), F32),
        ),
    )(u, Wv, bv, Wo, bo, W_la, b_la, pred0, g, mask2d)

    return (onehot, pred)


# speedup vs baseline: 3.6690x; 1.0132x over previous
import jax
import jax.numpy as jnp
from jax.experimental import pallas as pl

B, S, E, KD, VD, H, O = 64, 2048, 1024, 512, 512, 16, 20
DH = E // H
F32 = jnp.float32


def _setup_body(node_ref, Wlin_ref, blin_ref, Wq_ref, bq_ref, Wk_ref,
                p_ref, pred0_ref):
    x = node_ref[...]
    pred0_ref[...] = (
        jnp.dot(x, Wlin_ref[...].T, preferred_element_type=F32)
        + blin_ref[...][None, :])
    q = jnp.dot(x, Wq_ref[...].T, preferred_element_type=F32) + bq_ref[...][None, :]
    q = q * jnp.float32(1.0 / 8.0)
    for h in range(H):
        qh = q[:, h * DH:(h + 1) * DH]
        wk_h = Wk_ref[h * DH:(h + 1) * DH, :]
        p_ref[:, h, :] = jnp.dot(qh, wk_h, preferred_element_type=F32)


def _attn_body(p_ref, amask_ref, kpm_ref, seq_ref, u_ref):
    seq = seq_ref[0]
    seq_bf = seq.astype(jnp.bfloat16)
    pb = p_ref[0].astype(jnp.bfloat16)
    scores = jax.lax.dot_general(
        seq_bf, pb, (((1,), (1,)), ((), ())),
        preferred_element_type=F32)
    am = amask_ref[0, 0, :]
    kpm = kpm_ref[0, 0, :]
    scores = scores + am[:, None]
    scores = jnp.where(kpm[:, None] > 0, jnp.float32(-1e9), scores)
    m = jnp.max(scores, axis=0, keepdims=True)
    e = jnp.exp(scores - m)
    l = jnp.sum(e, axis=0, keepdims=True)
    a = (e / l).astype(jnp.bfloat16)
    u_ref[0] = jax.lax.dot_general(
        a, seq_bf, (((0,), (0,)), ((), ())),
        preferred_element_type=F32)


def _final_body(u_ref, Wv_ref, bv_ref, Wo_ref, bo_ref, Wla_ref, bla_ref,
                pred0_ref, g_ref, mask_ref, onehot_ref, pred_ref):
    ctx_parts = []
    for h in range(H):
        uh = u_ref[:, h, :]
        wv_h = Wv_ref[h * DH:(h + 1) * DH, :]
        ctx_parts.append(jax.lax.dot_general(
            uh, wv_h, (((1,), (1,)), ((), ())),
            preferred_element_type=F32))
    ctx = jnp.concatenate(ctx_parts, axis=1) + bv_ref[...][None, :]
    hout = jnp.dot(ctx, Wo_ref[...].T, preferred_element_type=F32) + bo_ref[...][None, :]
    pred = (pred0_ref[...]
            + jnp.dot(hout, Wla_ref[...].T, preferred_element_type=F32)
            + bla_ref[...][None, :])
    y = pred + g_ref[...]
    top = jnp.argmax(y, axis=-1)
    onehot = (jax.lax.broadcasted_iota(jnp.int32, (B, O), 1)
              == top[:, None]).astype(F32)
    msk = mask_ref[...]
    pred_ref[...] = pred * msk
    onehot_ref[...] = onehot * msk


def kernel(node_emb, seq_emb, mask, key_padding_mask, attn_mask,
           W_lin, b_lin, Wq, Wk, Wv, bq, bk, bv, Wo, bo, W_la, b_la):
    del bk
    x = node_emb.reshape(B, E)
    kpm_f = key_padding_mask.astype(F32).reshape(B, 1, S)
    amask = attn_mask.reshape(B, 1, S)
    mask2d = mask.reshape(B, 1)

    p, pred0 = pl.pallas_call(
        _setup_body,
        out_shape=(
            jax.ShapeDtypeStruct((B, H, KD), F32),
            jax.ShapeDtypeStruct((B, O), F32),
        ),
    )(x, W_lin, b_lin, Wq, bq, Wk)

    u = pl.pallas_call(
        _attn_body,
        grid=(B,),
        in_specs=[
            pl.BlockSpec((1, H, KD), lambda b: (b, 0, 0)),
            pl.BlockSpec((1, 1, S), lambda b: (b, 0, 0)),
            pl.BlockSpec((1, 1, S), lambda b: (b, 0, 0)),
            pl.BlockSpec((1, S, KD), lambda b: (b, 0, 0)),
        ],
        out_specs=pl.BlockSpec((1, H, KD), lambda b: (b, 0, 0)),
        out_shape=jax.ShapeDtypeStruct((B, H, KD), F32),
    )(p, amask, kpm_f, seq_emb)

    skey = jax.random.fold_in(jax.random.key(0), 12345)
    g = jax.random.gumbel(skey, (B, O), F32)

    onehot, pred = pl.pallas_call(
        _final_body,
        out_shape=(
            jax.ShapeDtypeStruct((B, O), F32),
            jax.ShapeDtypeStruct((B, O), F32),
        ),
    )(u, Wv, bv, Wo, bo, W_la, b_la, pred0, g, mask2d)

    return (onehot, pred)
